# baseline (speedup 1.0000x reference)
import functools

import jax
import jax.numpy as jnp
from jax import lax
from jax.experimental import pallas as pl
from jax.experimental.pallas import tpu as pltpu
from jax.experimental.pallas import tpu_sc as plsc

EMBED = 32
LANES = 16
NUM_CORES = 2
NUM_SUBCORES = 16
NUM_WORKERS = NUM_CORES * NUM_SUBCORES
DEPTH = 4


def kernel(user_indices, item_indices, user_table, item_table):
    batch = user_indices.shape[0]
    bpw = batch // NUM_WORKERS
    n_rows = user_table.shape[0]

    mesh = plsc.VectorSubcoreMesh(
        core_axis_name="c", subcore_axis_name="s",
        num_cores=NUM_CORES, num_subcores=NUM_SUBCORES)

    scratch = [
        pltpu.VMEM((bpw + LANES,), jnp.int32),
        pltpu.VMEM((bpw + LANES,), jnp.int32),
        pltpu.VMEM((2, 8, 128), jnp.float32),
        pltpu.VMEM((bpw,), jnp.float32),
    ]
    scratch += [pltpu.VMEM((4, 8, 128), jnp.float32) for _ in range(2 * DEPTH)]
    scratch += [pltpu.SemaphoreType.DMA for _ in range(2 * DEPTH)]

    @functools.partial(
        pl.kernel,
        mesh=mesh,
        out_type=jax.ShapeDtypeStruct((batch,), jnp.float32),
        scratch_types=scratch,
        compiler_params=pltpu.CompilerParams(needs_layout_passes=False),
    )
    def sc_kernel(uidx_hbm, iidx_hbm, utab_hbm, itab_hbm, out_hbm,
                  uidx_v, iidx_v, staged_v, out_v, *bufs_and_sems):
        ubufs = bufs_and_sems[0:DEPTH]
        ibufs = bufs_and_sems[DEPTH:2 * DEPTH]
        usems = bufs_and_sems[2 * DEPTH:3 * DEPTH]
        isems = bufs_and_sems[3 * DEPTH:4 * DEPTH]

        wid = lax.axis_index("s") * NUM_CORES + lax.axis_index("c")
        base = wid * bpw

        pltpu.sync_copy(uidx_hbm.at[pl.ds(base, bpw)], uidx_v.at[pl.ds(0, bpw)])
        pltpu.sync_copy(iidx_hbm.at[pl.ds(base, bpw)], iidx_v.at[pl.ds(0, bpw)])
        zeros16 = jnp.zeros((LANES,), jnp.int32)
        uidx_v[pl.ds(bpw, LANES)] = zeros16
        iidx_v[pl.ds(bpw, LANES)] = zeros16

        d16 = lax.iota(jnp.int32, LANES)
        bv = d16 >> 3
        sv = d16 & 7
        kb = d16 >> 3
        ks = d16 & 7

        def fire(slot, ru, ri):
            rbu = pl.multiple_of((ru >> 7) << 7, 128)
            rbi = pl.multiple_of((ri >> 7) << 7, 128)
            pltpu.async_copy(
                utab_hbm.at[:, :, pl.ds(rbu, 128)], ubufs[slot], usems[slot])
            pltpu.async_copy(
                itab_hbm.at[:, :, pl.ds(rbi, 128)], ibufs[slot], isems[slot])

        def wait(slot):
            pltpu.make_async_copy(
                utab_hbm.at[:, :, pl.ds(0, 128)], ubufs[slot],
                usems[slot]).wait()
            pltpu.make_async_copy(
                itab_hbm.at[:, :, pl.ds(0, 128)], ibufs[slot],
                isems[slot]).wait()

        uvec_p = uidx_v[pl.ds(0, LANES)]
        ivec_p = iidx_v[pl.ds(0, LANES)]
        for j in range(DEPTH):
            fire(j, uvec_p[j], ivec_p[j])

        def block_body(blk, carry):
            j0 = blk * LANES
            uvec_a = uidx_v[pl.ds(j0, LANES)]
            ivec_a = iidx_v[pl.ds(j0, LANES)]
            uvec_b = uidx_v[pl.ds(j0 + LANES, LANES)]
            ivec_b = iidx_v[pl.ds(j0 + LANES, LANES)]
            for k in range(LANES):
                j = j0 + k
                slot = k % DEPTH
                wait(slot)
                ru = uvec_a[k]
                ri = ivec_a[k]
                lu = jnp.full((LANES,), ru & 127, jnp.int32)
                li = jnp.full((LANES,), ri & 127, jnp.int32)
                u0 = plsc.load_gather(ubufs[slot], [bv, sv, lu])
                u1 = plsc.load_gather(ubufs[slot], [bv + 2, sv, lu])
                i0 = plsc.load_gather(ibufs[slot], [bv, sv, li])
                i1 = plsc.load_gather(ibufs[slot], [bv + 2, sv, li])
                p = u0 * i0 + u1 * i1
                staged_v[k // 8, k % 8, pl.ds(0, LANES)] = p

                if k + DEPTH < LANES:
                    ru_n = uvec_a[k + DEPTH]
                    ri_n = ivec_a[k + DEPTH]
                else:
                    ru_n = uvec_b[k + DEPTH - LANES]
                    ri_n = ivec_b[k + DEPTH - LANES]

                @pl.when(j + DEPTH < bpw)
                def _():
                    fire(slot, ru_n, ri_n)

            acc = jnp.zeros((LANES,), jnp.float32)
            for l in range(LANES):
                lv = jnp.full((LANES,), l, jnp.int32)
                acc = acc + plsc.load_gather(staged_v, [kb, ks, lv])
            out_v[pl.ds(j0, LANES)] = acc
            return carry

        lax.fori_loop(0, bpw // LANES, block_body, 0)
        pltpu.sync_copy(out_v, out_hbm.at[pl.ds(base, bpw)])

    ut3 = user_table.T.reshape(4, 8, n_rows)
    it3 = item_table.T.reshape(4, 8, n_rows)
    return sc_kernel(user_indices, item_indices, ut3, it3)


# speedup vs baseline: 1.4990x; 1.4990x over previous
import functools

import jax
import jax.numpy as jnp
from jax import lax
from jax.experimental import pallas as pl
from jax.experimental.pallas import tpu as pltpu
from jax.experimental.pallas import tpu_sc as plsc

EMBED = 32
LANES = 16
NUM_CORES = 2
NUM_SUBCORES = 16
NUM_WORKERS = NUM_CORES * NUM_SUBCORES
DEPTH = 8


def kernel(user_indices, item_indices, user_table, item_table):
    batch = user_indices.shape[0]
    bpw = batch // NUM_WORKERS
    n_rows = user_table.shape[0]

    mesh = plsc.VectorSubcoreMesh(
        core_axis_name="c", subcore_axis_name="s",
        num_cores=NUM_CORES, num_subcores=NUM_SUBCORES)

    scratch = [
        pltpu.VMEM((bpw + LANES,), jnp.int32),
        pltpu.VMEM((bpw + LANES,), jnp.int32),
        pltpu.VMEM((2, 8, 128), jnp.float32),
        pltpu.VMEM((bpw,), jnp.float32),
    ]
    scratch += [pltpu.VMEM((4, 8, 128), jnp.float32) for _ in range(2 * DEPTH)]
    scratch += [pltpu.SemaphoreType.DMA for _ in range(2 * DEPTH)]

    @functools.partial(
        pl.kernel,
        mesh=mesh,
        out_type=jax.ShapeDtypeStruct((batch,), jnp.float32),
        scratch_types=scratch,
        compiler_params=pltpu.CompilerParams(needs_layout_passes=False),
    )
    def sc_kernel(uidx_hbm, iidx_hbm, utab_hbm, itab_hbm, out_hbm,
                  uidx_v, iidx_v, staged_v, out_v, *bufs_and_sems):
        ubufs = bufs_and_sems[0:DEPTH]
        ibufs = bufs_and_sems[DEPTH:2 * DEPTH]
        usems = bufs_and_sems[2 * DEPTH:3 * DEPTH]
        isems = bufs_and_sems[3 * DEPTH:4 * DEPTH]

        wid = lax.axis_index("s") * NUM_CORES + lax.axis_index("c")
        base = wid * bpw

        pltpu.sync_copy(uidx_hbm.at[pl.ds(base, bpw)], uidx_v.at[pl.ds(0, bpw)])
        pltpu.sync_copy(iidx_hbm.at[pl.ds(base, bpw)], iidx_v.at[pl.ds(0, bpw)])
        zeros16 = jnp.zeros((LANES,), jnp.int32)
        uidx_v[pl.ds(bpw, LANES)] = zeros16
        iidx_v[pl.ds(bpw, LANES)] = zeros16

        d16 = lax.iota(jnp.int32, LANES)
        bv = d16 >> 3
        sv = d16 & 7
        kb = d16 >> 3
        ks = d16 & 7

        def fire(slot, ru, ri):
            rbu = pl.multiple_of((ru >> 7) << 7, 128)
            rbi = pl.multiple_of((ri >> 7) << 7, 128)
            lgu = pl.multiple_of(((ru & 127) >> 4) << 4, 16)
            lgi = pl.multiple_of(((ri & 127) >> 4) << 4, 16)
            pltpu.async_copy(
                utab_hbm.at[:, :, pl.ds(rbu, 128)].at[:, :, pl.ds(lgu, 16)], ubufs[slot].at[:, :, pl.ds(0, 16)], usems[slot])
            pltpu.async_copy(
                itab_hbm.at[:, :, pl.ds(rbi, 128)].at[:, :, pl.ds(lgi, 16)], ibufs[slot].at[:, :, pl.ds(0, 16)], isems[slot])

        def wait(slot):
            pltpu.make_async_copy(
                utab_hbm.at[:, :, pl.ds(0, 16)], ubufs[slot].at[:, :, pl.ds(0, 16)],
                usems[slot]).wait()
            pltpu.make_async_copy(
                itab_hbm.at[:, :, pl.ds(0, 16)], ibufs[slot].at[:, :, pl.ds(0, 16)],
                isems[slot]).wait()

        uvec_p = uidx_v[pl.ds(0, LANES)]
        ivec_p = iidx_v[pl.ds(0, LANES)]
        for j in range(DEPTH):
            fire(j, uvec_p[j], ivec_p[j])

        def block_body(blk, carry):
            j0 = blk * LANES
            uvec_a = uidx_v[pl.ds(j0, LANES)]
            ivec_a = iidx_v[pl.ds(j0, LANES)]
            uvec_b = uidx_v[pl.ds(j0 + LANES, LANES)]
            ivec_b = iidx_v[pl.ds(j0 + LANES, LANES)]
            for k in range(LANES):
                j = j0 + k
                slot = k % DEPTH
                wait(slot)
                ru = uvec_a[k]
                ri = ivec_a[k]
                lu = jnp.full((LANES,), ru & 15, jnp.int32)
                li = jnp.full((LANES,), ri & 15, jnp.int32)
                u0 = plsc.load_gather(ubufs[slot], [bv, sv, lu])
                u1 = plsc.load_gather(ubufs[slot], [bv + 2, sv, lu])
                i0 = plsc.load_gather(ibufs[slot], [bv, sv, li])
                i1 = plsc.load_gather(ibufs[slot], [bv + 2, sv, li])
                p = u0 * i0 + u1 * i1
                staged_v[k // 8, k % 8, pl.ds(0, LANES)] = p

                if k + DEPTH < LANES:
                    ru_n = uvec_a[k + DEPTH]
                    ri_n = ivec_a[k + DEPTH]
                else:
                    ru_n = uvec_b[k + DEPTH - LANES]
                    ri_n = ivec_b[k + DEPTH - LANES]

                @pl.when(j + DEPTH < bpw)
                def _():
                    fire(slot, ru_n, ri_n)

            acc = jnp.zeros((LANES,), jnp.float32)
            for l in range(LANES):
                lv = jnp.full((LANES,), l, jnp.int32)
                acc = acc + plsc.load_gather(staged_v, [kb, ks, lv])
            out_v[pl.ds(j0, LANES)] = acc
            return carry

        lax.fori_loop(0, bpw // LANES, block_body, 0)
        pltpu.sync_copy(out_v, out_hbm.at[pl.ds(base, bpw)])

    ut3 = user_table.T.reshape(4, 8, n_rows)
    it3 = item_table.T.reshape(4, 8, n_rows)
    return sc_kernel(user_indices, item_indices, ut3, it3)
